# baseline (speedup 1.0000x reference)
import functools

import jax
import jax.numpy as jnp
from jax.experimental import pallas as pl
from jax.experimental.pallas import tpu as pltpu

_NUM_EXPERTS = 64
_TOP_K = 8
_TILE = 512


def _router_body(x_ref, w_ref, w_out_ref, idx_out_ref):
    x = x_ref[0]
    w = w_ref[...]
    logits = jax.lax.dot_general(
        x, w, (((1,), (1,)), ((), ())),
        preferred_element_type=jnp.float32)
    lt = logits.T
    t = lt.shape[1]
    row = jax.lax.broadcasted_iota(jnp.int32, lt.shape, 0)
    row8 = jax.lax.broadcasted_iota(jnp.int32, (_TOP_K, t), 0)
    work = lt
    sel = jnp.zeros(lt.shape, dtype=jnp.bool_)
    idx_t = jnp.zeros((_TOP_K, t), jnp.int32)
    max0 = None
    for j in range(_TOP_K):
        m = jnp.max(work, axis=0, keepdims=True)
        if j == 0:
            max0 = m
        amax = jnp.min(jnp.where(work == m, row, _NUM_EXPERTS),
                       axis=0, keepdims=True)
        hit = row == amax
        sel = jnp.logical_or(sel, hit)
        work = jnp.where(hit, -jnp.inf, work)
        idx_t = jnp.where(row8 == j, amax, idx_t)
    e = jnp.where(sel, jnp.exp(lt - max0), 0.0)
    denom = jnp.sum(e, axis=0, keepdims=True)
    w_out_ref[0] = (e / denom).T
    idx_out_ref[0] = idx_t.T


def kernel(input, W):
    b, s, d = input.shape
    e = W.shape[0]
    tile = _TILE
    grid = (b, s // tile)
    weights, idx = pl.pallas_call(
        _router_body,
        grid=grid,
        in_specs=[
            pl.BlockSpec((1, tile, d), lambda i, j: (i, j, 0)),
            pl.BlockSpec((e, d), lambda i, j: (0, 0)),
        ],
        out_specs=[
            pl.BlockSpec((1, tile, e), lambda i, j: (i, j, 0)),
            pl.BlockSpec((1, tile, _TOP_K), lambda i, j: (i, j, 0)),
        ],
        out_shape=[
            jax.ShapeDtypeStruct((b, s, e), jnp.float32),
            jax.ShapeDtypeStruct((b, s, _TOP_K), jnp.int32),
        ],
        compiler_params=pltpu.CompilerParams(
            dimension_semantics=("arbitrary", "arbitrary"),
        ),
    )(input, W)
    return weights, idx
